# baseline (speedup 1.0000x reference)
import jax
import jax.numpy as jnp
from jax import lax
from jax.experimental import pallas as pl
from jax.experimental.pallas import tpu as pltpu
from jax.experimental.pallas import tpu_sc as plsc

_NUM_EMB = 1000000
_DIM = 64
_SCALE = float(_DIM) ** 0.5

_NC, _NS = 2, 16
_NW = _NC * _NS
_B = 16384 * 50
_G = 128
_BPW = _B // _NW
_NG = _BPW // _G
_NBUF = 2


def _splat(x):
    return lax.broadcast_in_dim(x, (16,), ())


def _body(idx_hbm, w_hbm, out_hbm, rowidx, halfoff, gb, sb, *sems):
    si, sg, ss = sems[0], sems[1:1 + _NBUF], sems[1 + _NBUF:]
    wid = lax.axis_index("s") * _NC + lax.axis_index("c")
    pltpu.async_copy(idx_hbm.at[pl.ds(wid * _NG, _NG)], rowidx, si).wait()

    @pl.loop(0, _NG, unroll=4)
    def _pre(r):
        for k in range(_G // 16):
            sl = pl.ds(k * 16, 16)
            raw = rowidx[r, sl]
            halfoff[r, sl] = (raw & 1) << 6
            rowidx[r, sl] = raw >> 1

    iota = lax.iota(jnp.int32, 16)
    row_half = iota >> 1
    col_par = (iota & 1) << 6
    orow0 = wid * (_BPW // 2)

    def _process(b, g):
        for jb in range(_G // 16):
            rowv = _splat(jb * 16) + iota
            halfv = halfoff[g, pl.ds(jb * 16, 16)]
            orow = _splat(jb * 8) + row_half

            @pl.loop(0, _DIM)
            def _col(c):
                cs = _splat(c)
                val = plsc.load_gather(gb.at[b], [rowv, halfv + cs])
                plsc.store_scatter(sb.at[b], [orow, col_par + cs],
                                   val * _SCALE)

    for b in range(_NBUF):
        pltpu.async_copy(w_hbm.at[rowidx.at[b]], gb.at[b], sg[b])

    @pl.loop(0, _NG, step=_NBUF)
    def _grp(g0):
        for b in range(_NBUF):
            g = g0 + b
            pltpu.make_async_copy(w_hbm.at[rowidx.at[b]], gb.at[b],
                                  sg[b]).wait()
            @pl.when(g0 > 0)
            def _():
                pltpu.make_async_copy(
                    sb.at[b], out_hbm.at[pl.ds(orow0, _G // 2)], ss[b]).wait()
            _process(b, g)
            @pl.when(g0 < _NG - _NBUF)
            def _():
                pltpu.async_copy(w_hbm.at[rowidx.at[g + _NBUF]], gb.at[b],
                                 sg[b])
            pltpu.async_copy(
                sb.at[b],
                out_hbm.at[pl.ds(orow0 + g * (_G // 2), _G // 2)], ss[b])

    for b in range(_NBUF):
        pltpu.make_async_copy(sb.at[b], out_hbm.at[pl.ds(orow0, _G // 2)],
                              ss[b]).wait()


@jax.jit
def _emb(idx2d, w128):
    mesh = plsc.VectorSubcoreMesh(core_axis_name="c", subcore_axis_name="s")
    return pl.kernel(
        _body,
        out_type=jax.ShapeDtypeStruct((_B // 2, 128), jnp.float32),
        mesh=mesh,
        compiler_params=pltpu.CompilerParams(needs_layout_passes=False),
        scratch_types=(
            [pltpu.VMEM((_NG, _G), jnp.int32),
             pltpu.VMEM((_NG, _G), jnp.int32),
             pltpu.VMEM((_NBUF, _G, 128), jnp.float32),
             pltpu.VMEM((_NBUF, _G // 2, 128), jnp.float32)]
            + [pltpu.SemaphoreType.DMA] * (1 + 2 * _NBUF)
        ),
    )(idx2d, w128)


def kernel(input, weight):
    idx2d = input.reshape(_B // _G, _G).astype(jnp.int32)
    w128 = weight.reshape(_NUM_EMB // 2, 128)
    out = _emb(idx2d, w128)
    return out.reshape(input.shape[0], input.shape[1], _DIM)


# speedup vs baseline: 2.4938x; 2.4938x over previous
import jax
import jax.numpy as jnp
from jax import lax
from jax.experimental import pallas as pl
from jax.experimental.pallas import tpu as pltpu
from jax.experimental.pallas import tpu_sc as plsc

_NUM_EMB = 1000000
_DIM = 64
_SCALE = float(_DIM) ** 0.5

_NC, _NS = 2, 16
_NW = _NC * _NS
_BATCH, _SEQ = 16384, 50
_B = _BATCH * _SEQ
_NBLK = 7812
_W2ROWS = 500000
_G = 128
_NGRP = _B // _G
_GPW = _NGRP // _NW
_K1_IT = 246


def _splat(x):
    return lax.broadcast_in_dim(x, (16,), ())


def _k1_body(wt_hbm, w2_hbm, vin, vout, vtin, vtout, *sems):
    sg, ss = sems[:2], sems[2:]
    wid = lax.axis_index("s") * _NC + lax.axis_index("c")
    iota = lax.iota(jnp.int32, 16)
    drow = (iota >> 1)
    dcol = (iota & 1) << 6

    def blk_of(i):
        return jnp.minimum(wid + 32 * i, _NBLK - 1)

    def transpose_scale(b):
        for eb in range(8):
            srccol = _splat(eb * 16) + iota
            dstrow = _splat(eb * 8) + drow
            for fb in range(4):
                @pl.loop(0, 16)
                def _d(d):
                    rot = (iota + _splat(d)) & 15
                    val = plsc.load_gather(
                        vin.at[b], [_splat(fb * 16) + rot, srccol])
                    plsc.store_scatter(
                        vout.at[b], [dstrow, dcol + _splat(fb * 16) + rot],
                        val * _SCALE)

    for b in range(2):
        pltpu.async_copy(
            wt_hbm.at[:, pl.ds(blk_of(b) * 128, 128)], vin.at[b], sg[b])

    @pl.loop(0, _K1_IT, step=2)
    def _it(i):
        for b in range(2):
            pltpu.make_async_copy(
                wt_hbm.at[:, pl.ds(0, 128)], vin.at[b], sg[b]).wait()
            @pl.when(i > 0)
            def _():
                pltpu.make_async_copy(
                    vout.at[b], w2_hbm.at[pl.ds(0, 64)], ss[b]).wait()
            transpose_scale(b)
            @pl.when(i < _K1_IT - 2)
            def _():
                pltpu.async_copy(
                    wt_hbm.at[:, pl.ds(blk_of(i + b + 2) * 128, 128)],
                    vin.at[b], sg[b])
            pltpu.async_copy(
                vout.at[b], w2_hbm.at[pl.ds(blk_of(i + b) * 64, 64)], ss[b])

    for b in range(2):
        pltpu.make_async_copy(
            vout.at[b], w2_hbm.at[pl.ds(0, 64)], ss[b]).wait()

    @pl.when(wid == 0)
    def _tail():
        pltpu.async_copy(
            wt_hbm.at[:, pl.ds(_NBLK * 128, 64)], vtin, sg[0]).wait()
        for eb in range(4):
            srccol = _splat(eb * 16) + iota
            dstrow = _splat(eb * 8) + drow
            for fb in range(4):
                @pl.loop(0, 16)
                def _d(d):
                    rot = (iota + _splat(d)) & 15
                    val = plsc.load_gather(
                        vtin, [_splat(fb * 16) + rot, srccol])
                    plsc.store_scatter(
                        vtout, [dstrow, dcol + _splat(fb * 16) + rot],
                        val * _SCALE)
        pltpu.async_copy(
            vtout, w2_hbm.at[pl.ds(_NBLK * 64, 32)], ss[0]).wait()


def _k2_body(idx_hbm, w2_hbm, out_hbm, rowidx, halfoff, gb, sb, *sems):
    si, sg, ss = sems[0], sems[1:3], sems[3:]
    wid = lax.axis_index("s") * _NC + lax.axis_index("c")
    pltpu.async_copy(
        idx_hbm.at[pl.ds(wid * _GPW, _GPW)], rowidx, si).wait()

    @pl.loop(0, _GPW, unroll=4)
    def _pre(r):
        for k in range(_G // 16):
            sl = pl.ds(k * 16, 16)
            raw = rowidx[r, sl]
            halfoff[r, sl] = (raw & 1) << 6
            rowidx[r, sl] = raw >> 1

    iota = lax.iota(jnp.int32, 16)
    gg0 = wid * _GPW

    def select_transpose(b, g):
        for jb in range(8):
            rowv = _splat(jb * 16) + iota
            halfv = halfoff[g, pl.ds(jb * 16, 16)]
            for fb in range(4):
                hc = halfv + _splat(fb * 16)
                @pl.loop(0, 16)
                def _d(d):
                    rot = (iota + _splat(d)) & 15
                    val = plsc.load_gather(gb.at[b], [rowv, hc + rot])
                    plsc.store_scatter(
                        sb.at[b], [_splat(fb * 16) + rot, rowv], val)

    for b in range(2):
        pltpu.async_copy(w2_hbm.at[rowidx.at[b]], gb.at[b], sg[b])

    @pl.loop(0, _GPW, step=2)
    def _grp(g0):
        for b in range(2):
            g = g0 + b
            pltpu.make_async_copy(
                w2_hbm.at[rowidx.at[b]], gb.at[b], sg[b]).wait()
            @pl.when(g0 > 0)
            def _():
                pltpu.make_async_copy(
                    sb.at[b], out_hbm.at[0, :, pl.ds(0, _G)], ss[b]).wait()
            select_transpose(b, g)
            @pl.when(g0 < _GPW - 2)
            def _():
                pltpu.async_copy(w2_hbm.at[rowidx.at[g + 2]], gb.at[b], sg[b])
            gg = gg0 + g
            pltpu.async_copy(
                sb.at[b],
                out_hbm.at[gg >> 7, :, pl.ds((gg & 127) * _G, _G)], ss[b])

    for b in range(2):
        pltpu.make_async_copy(
            sb.at[b], out_hbm.at[0, :, pl.ds(0, _G)], ss[b]).wait()


@jax.jit
def _emb(idx_flat, wt):
    mesh = plsc.VectorSubcoreMesh(core_axis_name="c", subcore_axis_name="s")
    cp = pltpu.CompilerParams(needs_layout_passes=False)
    w2 = pl.kernel(
        _k1_body,
        out_type=jax.ShapeDtypeStruct((_W2ROWS, 128), jnp.float32),
        mesh=mesh,
        compiler_params=cp,
        scratch_types=(
            [pltpu.VMEM((2, _DIM, 128), jnp.float32),
             pltpu.VMEM((2, _DIM, 128), jnp.float32),
             pltpu.VMEM((_DIM, 64), jnp.float32),
             pltpu.VMEM((32, 128), jnp.float32)]
            + [pltpu.SemaphoreType.DMA] * 4
        ),
    )(wt)
    out3 = pl.kernel(
        _k2_body,
        out_type=jax.ShapeDtypeStruct((_SEQ, _DIM, _BATCH), jnp.float32),
        mesh=mesh,
        compiler_params=cp,
        scratch_types=(
            [pltpu.VMEM((_GPW, _G), jnp.int32),
             pltpu.VMEM((_GPW, _G), jnp.int32),
             pltpu.VMEM((2, _G, 128), jnp.float32),
             pltpu.VMEM((2, _DIM, _G), jnp.float32)]
            + [pltpu.SemaphoreType.DMA] * 5
        ),
    )(idx_flat.reshape(_NGRP, _G), w2)
    return out3


def kernel(input, weight):
    idx_flat = input.astype(jnp.int32).T.reshape(_B)
    out3 = _emb(idx_flat, weight.T)
    return out3.transpose(2, 0, 1)
